# baseline (speedup 1.0000x reference)
import functools

import jax
import jax.numpy as jnp
from jax import lax
from jax.experimental import pallas as pl
from jax.experimental.pallas import tpu as pltpu
from jax.experimental.pallas import tpu_sc as plsc

NUM_FIELDS = 26
VOCAB = 100000
HIDDEN = 64
BATCH = 16384

BF = BATCH * NUM_FIELDS
CHUNK = 128
NC = 2
NS = 16
NW = NC * NS
N_CHUNKS = BF // CHUNK
CPW = N_CHUNKS // NW
ROWW = 2 * HIDDEN

_MESH = plsc.VectorSubcoreMesh(core_axis_name="c", subcore_axis_name="s")


_VBT = 8192


def _tp_body(in_ref, o_ref):
    x = in_ref[...]
    eye2 = jnp.concatenate(
        [jnp.eye(HIDDEN, dtype=jnp.float32)] * 2, axis=1)
    t2 = lax.dot_general(x, eye2, (((0,), (0,)), ((), ())),
                         preferred_element_type=jnp.float32)
    o_ref[0] = t2.astype(jnp.bfloat16)


def _transpose_tables(tab_t):
    out3 = pl.pallas_call(
        _tp_body,
        grid=(NUM_FIELDS, pl.cdiv(VOCAB, _VBT)),
        in_specs=[pl.BlockSpec((HIDDEN, _VBT), lambda i, v: (i, v))],
        out_specs=pl.BlockSpec((1, _VBT, ROWW), lambda i, v: (i, v, 0)),
        out_shape=jax.ShapeDtypeStruct((NUM_FIELDS, VOCAB, ROWW),
                                       jnp.bfloat16),
    )(tab_t)
    return out3.reshape(NUM_FIELDS * VOCAB, ROWW)


@functools.partial(
    pl.kernel,
    mesh=_MESH,
    out_type=jax.ShapeDtypeStruct((BF, HIDDEN), jnp.bfloat16),
    scratch_types=[
        pltpu.VMEM((CPW, CHUNK), jnp.int32),
        pltpu.VMEM((CHUNK, ROWW), jnp.bfloat16),
        pltpu.SemaphoreType.DMA,
    ],
    compiler_params=pltpu.CompilerParams(use_tc_tiling_on_sc=False),
)
def _sc_gather(tab_hbm, idx_hbm, out_hbm, idx_v, rows_v, gsem):
    wid = lax.axis_index("s") * NC + lax.axis_index("c")
    cbase = wid * CPW
    pltpu.sync_copy(idx_hbm.at[pl.ds(cbase, CPW)], idx_v)

    def body(j, carry):
        pltpu.async_copy(tab_hbm.at[idx_v.at[j]], rows_v, gsem).wait()
        pltpu.sync_copy(rows_v.at[:, pl.ds(0, HIDDEN)],
                        out_hbm.at[pl.ds((cbase + j) * CHUNK, CHUNK)])
        return carry

    lax.fori_loop(0, CPW, body, 0)


_BM = 1024


def _mm_body(g_ref, w_ref, b_ref, o_ref):
    k = pl.program_id(1)
    acc = jnp.dot(g_ref[0], w_ref[0], preferred_element_type=jnp.float32)

    @pl.when(k == 0)
    def _init():
        o_ref[...] = acc + b_ref[...]

    @pl.when(k != 0)
    def _acc():
        o_ref[...] += acc


def _tc_project(g3, Wd, b2):
    return pl.pallas_call(
        _mm_body,
        grid=(BATCH // 2 // _BM, NUM_FIELDS),
        in_specs=[
            pl.BlockSpec((1, _BM, ROWW), lambda i, k: (k, i, 0)),
            pl.BlockSpec((1, ROWW, ROWW), lambda i, k: (k, 0, 0)),
            pl.BlockSpec((1, ROWW), lambda i, k: (0, 0)),
        ],
        out_specs=pl.BlockSpec((_BM, ROWW), lambda i, k: (i, 0)),
        out_shape=jax.ShapeDtypeStruct((BATCH // 2, ROWW), jnp.float32),
    )(g3, Wd, b2.reshape(1, ROWW))


def kernel(x, tables, W, b):
    tab_t = tables.transpose(0, 2, 1).reshape(NUM_FIELDS * HIDDEN, VOCAB)
    tab_flat = _transpose_tables(tab_t)

    offs = jnp.arange(NUM_FIELDS, dtype=jnp.int32) * VOCAB
    idx = (x.T.astype(jnp.int32) + offs[:, None]).reshape(N_CHUNKS, CHUNK)

    gathered = _sc_gather(tab_flat, idx)
    g3 = gathered.reshape(NUM_FIELDS, BATCH // 2, ROWW)

    Wr = W.reshape(NUM_FIELDS, HIDDEN, HIDDEN)
    Wd = jnp.zeros((NUM_FIELDS, ROWW, ROWW), jnp.float32)
    Wd = Wd.at[:, :HIDDEN, :HIDDEN].set(Wr).at[:, HIDDEN:, HIDDEN:].set(Wr)
    Wd = Wd.astype(jnp.bfloat16)
    b2 = jnp.concatenate([b, b])

    out_pairs = _tc_project(g3, Wd, b2)
    return out_pairs.reshape(BATCH, HIDDEN)


# speedup vs baseline: 2.9148x; 2.9148x over previous
import functools

import jax
import jax.numpy as jnp
from jax import lax
from jax.experimental import pallas as pl
from jax.experimental.pallas import tpu as pltpu
from jax.experimental.pallas import tpu_sc as plsc

NUM_FIELDS = 26
VOCAB = 100000
HIDDEN = 64
BATCH = 16384

BF = BATCH * NUM_FIELDS
CHUNK = 128
NC = 2
NS = 16
NW = NC * NS
N_CHUNKS = BF // CHUNK
CPW = N_CHUNKS // NW
ROWW = 2 * HIDDEN

_MESH = plsc.VectorSubcoreMesh(core_axis_name="c", subcore_axis_name="s")


_VBT = 8192


def _tp_body(in_ref, o_ref):
    x = in_ref[...]
    eye2 = jnp.concatenate(
        [jnp.eye(HIDDEN, dtype=jnp.float32)] * 2, axis=1)
    t2 = lax.dot_general(x, eye2, (((0,), (0,)), ((), ())),
                         preferred_element_type=jnp.float32)
    o_ref[0] = t2


def _transpose_tables(tab_t):
    out3 = pl.pallas_call(
        _tp_body,
        grid=(NUM_FIELDS, pl.cdiv(VOCAB, _VBT)),
        in_specs=[pl.BlockSpec((HIDDEN, _VBT), lambda i, v: (i, v))],
        out_specs=pl.BlockSpec((1, _VBT, ROWW), lambda i, v: (i, v, 0)),
        out_shape=jax.ShapeDtypeStruct((NUM_FIELDS, VOCAB, ROWW),
                                       jnp.float32),
    )(tab_t)
    return out3.reshape(NUM_FIELDS * VOCAB, ROWW)


@functools.partial(
    pl.kernel,
    mesh=_MESH,
    out_type=jax.ShapeDtypeStruct((BF, HIDDEN), jnp.float32),
    scratch_types=[
        pltpu.VMEM((CPW, CHUNK), jnp.int32),
        pltpu.VMEM((CHUNK, ROWW), jnp.float32),
        pltpu.SemaphoreType.DMA,
    ],
    compiler_params=pltpu.CompilerParams(use_tc_tiling_on_sc=False),
)
def _sc_gather(tab_hbm, idx_hbm, out_hbm, idx_v, rows_v, gsem):
    wid = lax.axis_index("s") * NC + lax.axis_index("c")
    cbase = wid * CPW
    pltpu.sync_copy(idx_hbm.at[pl.ds(cbase, CPW)], idx_v)

    def body(j, carry):
        pltpu.async_copy(tab_hbm.at[idx_v.at[j]], rows_v, gsem).wait()
        pltpu.sync_copy(rows_v.at[:, pl.ds(0, HIDDEN)],
                        out_hbm.at[pl.ds((cbase + j) * CHUNK, CHUNK)])
        return carry

    lax.fori_loop(0, CPW, body, 0)


_BM = 1024


def _mm_body(g_ref, w_ref, b_ref, o_ref):
    k = pl.program_id(1)
    acc = jnp.dot(g_ref[0], w_ref[0], preferred_element_type=jnp.float32)

    @pl.when(k == 0)
    def _init():
        o_ref[...] = acc + b_ref[...]

    @pl.when(k != 0)
    def _acc():
        o_ref[...] += acc


def _tc_project(g3, Wd, b2):
    return pl.pallas_call(
        _mm_body,
        grid=(BATCH // 2 // _BM, NUM_FIELDS),
        in_specs=[
            pl.BlockSpec((1, _BM, ROWW), lambda i, k: (k, i, 0)),
            pl.BlockSpec((1, ROWW, ROWW), lambda i, k: (k, 0, 0)),
            pl.BlockSpec((1, ROWW), lambda i, k: (0, 0)),
        ],
        out_specs=pl.BlockSpec((_BM, ROWW), lambda i, k: (i, 0)),
        out_shape=jax.ShapeDtypeStruct((BATCH // 2, ROWW), jnp.float32),
    )(g3, Wd, b2.reshape(1, ROWW))


def kernel(x, tables, W, b):
    tab_t = tables.transpose(0, 2, 1).reshape(NUM_FIELDS * HIDDEN, VOCAB)
    tab_flat = _transpose_tables(tab_t)

    offs = jnp.arange(NUM_FIELDS, dtype=jnp.int32) * VOCAB
    idx = (x.T.astype(jnp.int32) + offs[:, None]).reshape(N_CHUNKS, CHUNK)

    gathered = _sc_gather(tab_flat, idx)
    g3 = gathered.reshape(NUM_FIELDS, BATCH // 2, ROWW)

    Wr = W.reshape(NUM_FIELDS, HIDDEN, HIDDEN)
    Wd = jnp.zeros((NUM_FIELDS, ROWW, ROWW), jnp.float32)
    Wd = Wd.at[:, :HIDDEN, :HIDDEN].set(Wr).at[:, HIDDEN:, HIDDEN:].set(Wr)
    b2 = jnp.concatenate([b, b])

    out_pairs = _tc_project(g3, Wd, b2)
    return out_pairs.reshape(BATCH, HIDDEN)


# speedup vs baseline: 3.3968x; 1.1654x over previous
import functools

import jax
import jax.numpy as jnp
from jax import lax
from jax.experimental import pallas as pl
from jax.experimental.pallas import tpu as pltpu
from jax.experimental.pallas import tpu_sc as plsc

NUM_FIELDS = 26
VOCAB = 100000
HIDDEN = 64
BATCH = 16384

BF = BATCH * NUM_FIELDS
CHUNK = 128
NC = 2
NS = 16
NW = NC * NS
N_CHUNKS = BF // CHUNK
CPW = N_CHUNKS // NW
ROWW = 2 * HIDDEN

_MESH = plsc.VectorSubcoreMesh(core_axis_name="c", subcore_axis_name="s")


_VBT = 8192


def _tp_body(in_ref, o_ref):
    x = in_ref[...]
    eye2 = jnp.concatenate(
        [jnp.eye(HIDDEN, dtype=jnp.float32)] * 2, axis=1)
    t2 = lax.dot_general(x, eye2, (((0,), (0,)), ((), ())),
                         preferred_element_type=jnp.float32)
    o_ref[0] = t2


def _transpose_tables(tab_t):
    out3 = pl.pallas_call(
        _tp_body,
        grid=(NUM_FIELDS, pl.cdiv(VOCAB, _VBT)),
        in_specs=[pl.BlockSpec((HIDDEN, _VBT), lambda i, v: (i, v))],
        out_specs=pl.BlockSpec((1, _VBT, ROWW), lambda i, v: (i, v, 0)),
        out_shape=jax.ShapeDtypeStruct((NUM_FIELDS, VOCAB, ROWW),
                                       jnp.float32),
    )(tab_t)
    return out3.reshape(NUM_FIELDS * VOCAB, ROWW)


@functools.partial(
    pl.kernel,
    mesh=_MESH,
    out_type=jax.ShapeDtypeStruct((BF, HIDDEN), jnp.float32),
    scratch_types=[
        pltpu.VMEM((CPW, CHUNK), jnp.int32),
        pltpu.VMEM((4, CHUNK, ROWW), jnp.float32),
        pltpu.SemaphoreType.DMA,
    ],
    compiler_params=pltpu.CompilerParams(use_tc_tiling_on_sc=False),
)
def _sc_gather(tab_hbm, idx_hbm, out_hbm, idx_v, rows_v, gsem):
    wid = lax.axis_index("s") * NC + lax.axis_index("c")
    cbase = wid * CPW
    pltpu.sync_copy(idx_hbm.at[pl.ds(cbase, CPW)], idx_v)

    def body(jj, carry):
        j0 = jj * 4
        cps = [
            pltpu.async_copy(tab_hbm.at[idx_v.at[j0 + u]], rows_v.at[u], gsem)
            for u in range(4)
        ]
        for cp in cps:
            cp.wait()
        for u in range(4):
            pltpu.sync_copy(
                rows_v.at[u, :, pl.ds(0, HIDDEN)],
                out_hbm.at[pl.ds((cbase + j0 + u) * CHUNK, CHUNK)])
        return carry

    lax.fori_loop(0, CPW // 4, body, 0)


_BM = 8192


def _mm_body(g_ref, w_ref, b_ref, o_ref):
    k = pl.program_id(1)
    acc = jnp.dot(g_ref[0], w_ref[0], preferred_element_type=jnp.float32)

    @pl.when(k == 0)
    def _init():
        o_ref[...] = acc + b_ref[...]

    @pl.when(k != 0)
    def _acc():
        o_ref[...] += acc


def _tc_project(g3, Wd, b2):
    return pl.pallas_call(
        _mm_body,
        grid=(BATCH // 2 // _BM, NUM_FIELDS),
        in_specs=[
            pl.BlockSpec((1, _BM, ROWW), lambda i, k: (k, i, 0)),
            pl.BlockSpec((1, ROWW, ROWW), lambda i, k: (k, 0, 0)),
            pl.BlockSpec((1, ROWW), lambda i, k: (0, 0)),
        ],
        out_specs=pl.BlockSpec((_BM, ROWW), lambda i, k: (i, 0)),
        out_shape=jax.ShapeDtypeStruct((BATCH // 2, ROWW), jnp.float32),
    )(g3, Wd, b2.reshape(1, ROWW))


def kernel(x, tables, W, b):
    tab_t = tables.transpose(0, 2, 1).reshape(NUM_FIELDS * HIDDEN, VOCAB)
    tab_flat = _transpose_tables(tab_t)

    offs = jnp.arange(NUM_FIELDS, dtype=jnp.int32) * VOCAB
    idx = (x.T.astype(jnp.int32) + offs[:, None]).reshape(N_CHUNKS, CHUNK)

    gathered = _sc_gather(tab_flat, idx)
    g3 = gathered.reshape(NUM_FIELDS, BATCH // 2, ROWW)

    Wr = W.reshape(NUM_FIELDS, HIDDEN, HIDDEN)
    Wd = jnp.einsum("pq,iab->ipaqb", jnp.eye(2, dtype=jnp.float32),
                    Wr).reshape(NUM_FIELDS, ROWW, ROWW)
    b2 = jnp.concatenate([b, b])

    out_pairs = _tc_project(g3, Wd, b2)
    return out_pairs.reshape(BATCH, HIDDEN)
